# Initial kernel scaffold; baseline (speedup 1.0000x reference)
import jax
import jax.numpy as jnp
from jax.experimental import pallas as pl


def kernel(x, embed):
    raise NotImplementedError("write your pallas kernel here")


# speedup vs baseline: 1.5744x; 1.5744x over previous
import functools

import jax
import jax.numpy as jnp
from jax import lax
from jax.experimental import pallas as pl
from jax.experimental.pallas import tpu as pltpu
from jax.experimental.pallas import tpu_sc as plsc

_N = 8192
_C = 8192
_D = 32
_TN = 256
_TC = 1024
_NI = _N // _TN
_NJ = _C // _TC
_INT_MAX = jnp.iinfo(jnp.int32).max


_TG = 256


def _dist_body(x_ref, et_ref, dist_ref, ind_ref, m_sc, a_sc):
    j = pl.program_id(1)
    xs = x_ref[...]
    es = et_ref[...]
    x2 = jnp.sum(xs * xs, axis=1, keepdims=True)
    y2 = jnp.sum(es * es, axis=0, keepdims=True)

    @pl.when(j == 0)
    def _():
        m_sc[...] = jnp.full((_TN, 128), jnp.inf, jnp.float32)
        a_sc[...] = jnp.zeros((_TN, 128), jnp.int32)

    lane = lax.broadcasted_iota(jnp.int32, (_TN, 128), 1)
    m_run = m_sc[...]
    a_run = a_sc[...]

    for g in range(_TC // _TG):
        es_g = es[:, g * _TG:(g + 1) * _TG]
        xy = lax.dot_general(
            xs, es_g, (((1,), (0,)), ((), ())),
            preferred_element_type=jnp.float32,
        )
        y2_g = y2[:, g * _TG:(g + 1) * _TG]
        sq = (x2 + y2_g) + xy * -2.0
        s = jnp.sqrt(jnp.maximum(sq, 0.0))
        dist_ref[:, g * _TG:(g + 1) * _TG] = -s
        for h in range(_TG // 128):
            s_h = s[:, h * 128:(h + 1) * 128]
            col = lane + (j * _TC + g * _TG + h * 128)
            upd = s_h < m_run
            a_run = jnp.where(upd, col, a_run)
            m_run = jnp.minimum(m_run, s_h)

    m_sc[...] = m_run
    a_sc[...] = a_run

    @pl.when(j == _NJ - 1)
    def _():
        rmin = jnp.min(m_run, axis=1, keepdims=True)
        cand = jnp.where(m_run == rmin, a_run, _INT_MAX)
        ind_ref[...] = jnp.min(cand, axis=1, keepdims=True)


def _dist_argmin(x2d, embed_t):
    return pl.pallas_call(
        _dist_body,
        grid=(_NI, _NJ),
        in_specs=[
            pl.BlockSpec((_TN, _D), lambda i, j: (i, 0)),
            pl.BlockSpec((_D, _TC), lambda i, j: (0, j)),
        ],
        out_specs=[
            pl.BlockSpec((_TN, _TC), lambda i, j: (i, j)),
            pl.BlockSpec((_TN, 1), lambda i, j: (i, 0)),
        ],
        out_shape=[
            jax.ShapeDtypeStruct((_N, _C), jnp.float32),
            jax.ShapeDtypeStruct((_N, 1), jnp.int32),
        ],
        scratch_shapes=[
            pltpu.VMEM((_TN, 128), jnp.float32),
            pltpu.VMEM((_TN, 128), jnp.int32),
        ],
    )(x2d, embed_t)


def _sc_gather(table, idx):
    info = plsc.get_sparse_core_info()
    nc, ns = info.num_cores, info.num_subcores
    nw = nc * ns
    b_per_w = _N // nw
    chunks = b_per_w // 128
    mesh = plsc.VectorSubcoreMesh(core_axis_name="c", subcore_axis_name="s")

    @functools.partial(
        pl.kernel,
        mesh=mesh,
        compiler_params=pltpu.CompilerParams(use_tc_tiling_on_sc=False),
        out_type=jax.ShapeDtypeStruct((_N, _D), jnp.float32),
        scratch_types=[
            pltpu.VMEM((chunks, 128), jnp.int32),
            pltpu.VMEM((chunks, 128, _D), jnp.float32),
            pltpu.SemaphoreType.DMA,
        ],
    )
    def gather_kernel(idx_hbm, table_hbm, out_hbm, idx_v, rows_v, sem):
        wid = lax.axis_index("s") * nc + lax.axis_index("c")
        base = wid * b_per_w
        for k in range(chunks):
            off = base + k * 128
            pltpu.sync_copy(idx_hbm.at[pl.ds(off, 128)], idx_v.at[k])
            pltpu.async_copy(table_hbm.at[idx_v.at[k]], rows_v.at[k], sem).wait()
            pltpu.sync_copy(rows_v.at[k], out_hbm.at[pl.ds(off, 128)])

    return gather_kernel(idx, table)


def kernel(x, embed):
    x = x.astype(jnp.float32)
    h = x.shape[0]
    x2d = x.reshape(_N, _D)
    e2d = embed.reshape(_C, _D)
    dist2d, ind2d = _dist_argmin(x2d, e2d.T)
    ind_flat = ind2d.reshape(_N)
    quant2d = _sc_gather(e2d, ind_flat)
    quantize = quant2d.reshape(h, _N, _D)
    embed_ind = ind_flat.reshape(h, _N)
    dist_u = dist2d.reshape(h, _N, _C)
    return (quantize, embed_ind, dist_u)


# speedup vs baseline: 2.3478x; 1.4913x over previous
import functools

import jax
import jax.numpy as jnp
from jax import lax
from jax.experimental import pallas as pl
from jax.experimental.pallas import tpu as pltpu
from jax.experimental.pallas import tpu_sc as plsc

_N = 8192
_C = 8192
_D = 32
_TN = 256
_TC = 8192
_NI = _N // _TN
_NJ = _C // _TC
_INT_MAX = jnp.iinfo(jnp.int32).max
_NJ_C = _TC // 128


_TG = 256


def _dist_body(x_ref, et_ref, dist_ref, ind_ref, m_sc, a_sc):
    j = pl.program_id(1)
    xs = x_ref[...]
    es = et_ref[...]
    x2 = jnp.sum(xs * xs, axis=1, keepdims=True)
    y2 = jnp.sum(es * es, axis=0, keepdims=True)

    @pl.when(j == 0)
    def _():
        m_sc[...] = jnp.full((_TN, 128), jnp.inf, jnp.float32)
        a_sc[...] = jnp.zeros((_TN, 128), jnp.int32)

    lane = lax.broadcasted_iota(jnp.int32, (_TN, 128), 1)
    m_run = m_sc[...]
    a_run = a_sc[...]

    for g in range(_TC // _TG):
        es_g = es[:, g * _TG:(g + 1) * _TG]
        xy = lax.dot_general(
            xs, es_g, (((1,), (0,)), ((), ())),
            preferred_element_type=jnp.float32,
        )
        y2_g = y2[:, g * _TG:(g + 1) * _TG]
        sq = (x2 + y2_g) + xy * -2.0
        s = jnp.sqrt(jnp.maximum(sq, 0.0))
        dist_ref[:, g * _TG:(g + 1) * _TG] = -s
        for h in range(_TG // 128):
            s_h = s[:, h * 128:(h + 1) * 128]
            upd = s_h < m_run
            a_run = jnp.where(upd, jnp.int32(j * _NJ_C + g * (_TG // 128) + h), a_run)
            m_run = jnp.minimum(m_run, s_h)

    m_sc[...] = m_run
    a_sc[...] = a_run

    @pl.when(j == _NJ - 1)
    def _():
        col = a_run * 128 + lane
        rmin = jnp.min(m_run, axis=1, keepdims=True)
        cand = jnp.where(m_run == rmin, col, _INT_MAX)
        ind_ref[...] = jnp.min(cand, axis=1, keepdims=True)


def _dist_argmin(x2d, embed_t):
    return pl.pallas_call(
        _dist_body,
        grid=(_NI, _NJ),
        in_specs=[
            pl.BlockSpec((_TN, _D), lambda i, j: (i, 0)),
            pl.BlockSpec((_D, _TC), lambda i, j: (0, j)),
        ],
        out_specs=[
            pl.BlockSpec((_TN, _TC), lambda i, j: (i, j)),
            pl.BlockSpec((_TN, 1), lambda i, j: (i, 0)),
        ],
        out_shape=[
            jax.ShapeDtypeStruct((_N, _C), jnp.float32),
            jax.ShapeDtypeStruct((_N, 1), jnp.int32),
        ],
        scratch_shapes=[
            pltpu.VMEM((_TN, 128), jnp.float32),
            pltpu.VMEM((_TN, 128), jnp.int32),
        ],
    )(x2d, embed_t)


def _sc_gather(table, idx):
    info = plsc.get_sparse_core_info()
    nc, ns = info.num_cores, info.num_subcores
    nw = nc * ns
    b_per_w = _N // nw
    chunks = b_per_w // 128
    mesh = plsc.VectorSubcoreMesh(core_axis_name="c", subcore_axis_name="s")

    @functools.partial(
        pl.kernel,
        mesh=mesh,
        compiler_params=pltpu.CompilerParams(use_tc_tiling_on_sc=False),
        out_type=jax.ShapeDtypeStruct((_N, _D), jnp.float32),
        scratch_types=[
            pltpu.VMEM((chunks, 128), jnp.int32),
            pltpu.VMEM((chunks, 128, _D), jnp.float32),
            pltpu.SemaphoreType.DMA,
        ],
    )
    def gather_kernel(idx_hbm, table_hbm, out_hbm, idx_v, rows_v, sem):
        wid = lax.axis_index("s") * nc + lax.axis_index("c")
        base = wid * b_per_w
        for k in range(chunks):
            off = base + k * 128
            pltpu.sync_copy(idx_hbm.at[pl.ds(off, 128)], idx_v.at[k])
            pltpu.async_copy(table_hbm.at[idx_v.at[k]], rows_v.at[k], sem).wait()
            pltpu.sync_copy(rows_v.at[k], out_hbm.at[pl.ds(off, 128)])

    return gather_kernel(idx, table)


def kernel(x, embed):
    x = x.astype(jnp.float32)
    h = x.shape[0]
    x2d = x.reshape(_N, _D)
    e2d = embed.reshape(_C, _D)
    dist2d, ind2d = _dist_argmin(x2d, e2d.T)
    ind_flat = ind2d.reshape(_N)
    quant2d = _sc_gather(e2d, ind_flat)
    quantize = quant2d.reshape(h, _N, _D)
    embed_ind = ind_flat.reshape(h, _N)
    dist_u = dist2d.reshape(h, _N, _C)
    return (quantize, embed_ind, dist_u)


# speedup vs baseline: 2.4458x; 1.0417x over previous
import functools

import jax
import jax.numpy as jnp
from jax import lax
from jax.experimental import pallas as pl
from jax.experimental.pallas import tpu as pltpu
from jax.experimental.pallas import tpu_sc as plsc

_N = 8192
_C = 8192
_D = 32
_TN = 512
_TC = 8192
_NI = _N // _TN
_NJ = _C // _TC
_INT_MAX = jnp.iinfo(jnp.int32).max
_NJ_C = _TC // 128


_TG = 256


def _dist_body(x_ref, et_ref, dist_ref, ind_ref, m_sc, a_sc):
    j = pl.program_id(1)
    xs = x_ref[...]
    es = et_ref[...]
    x2 = jnp.sum(xs * xs, axis=1, keepdims=True)
    y2 = jnp.sum(es * es, axis=0, keepdims=True)

    @pl.when(j == 0)
    def _():
        m_sc[...] = jnp.full((_TN, 128), jnp.inf, jnp.float32)
        a_sc[...] = jnp.zeros((_TN, 128), jnp.int32)

    lane = lax.broadcasted_iota(jnp.int32, (_TN, 128), 1)
    m_run = m_sc[...]
    a_run = a_sc[...]

    for g in range(_TC // _TG):
        es_g = es[:, g * _TG:(g + 1) * _TG]
        xy = lax.dot_general(
            xs, es_g, (((1,), (0,)), ((), ())),
            preferred_element_type=jnp.float32,
        )
        y2_g = y2[:, g * _TG:(g + 1) * _TG]
        sq = (x2 + y2_g) + xy * -2.0
        s = jnp.sqrt(jnp.maximum(sq, 0.0))
        dist_ref[:, g * _TG:(g + 1) * _TG] = -s
        for h in range(_TG // 128):
            s_h = s[:, h * 128:(h + 1) * 128]
            upd = s_h < m_run
            a_run = jnp.where(upd, jnp.int32(j * _NJ_C + g * (_TG // 128) + h), a_run)
            m_run = jnp.minimum(m_run, s_h)

    m_sc[...] = m_run
    a_sc[...] = a_run

    @pl.when(j == _NJ - 1)
    def _():
        col = a_run * 128 + lane
        rmin = jnp.min(m_run, axis=1, keepdims=True)
        cand = jnp.where(m_run == rmin, col, _INT_MAX)
        ind_ref[...] = jnp.min(cand, axis=1, keepdims=True)


def _dist_argmin(x2d, embed_t):
    return pl.pallas_call(
        _dist_body,
        grid=(_NI, _NJ),
        in_specs=[
            pl.BlockSpec((_TN, _D), lambda i, j: (i, 0)),
            pl.BlockSpec((_D, _TC), lambda i, j: (0, j)),
        ],
        out_specs=[
            pl.BlockSpec((_TN, _TC), lambda i, j: (i, j)),
            pl.BlockSpec((_TN, 1), lambda i, j: (i, 0)),
        ],
        out_shape=[
            jax.ShapeDtypeStruct((_N, _C), jnp.float32),
            jax.ShapeDtypeStruct((_N, 1), jnp.int32),
        ],
        scratch_shapes=[
            pltpu.VMEM((_TN, 128), jnp.float32),
            pltpu.VMEM((_TN, 128), jnp.int32),
        ],
    )(x2d, embed_t)


def _sc_gather(table, idx):
    info = plsc.get_sparse_core_info()
    nc, ns = info.num_cores, info.num_subcores
    nw = nc * ns
    b_per_w = _N // nw
    chunks = b_per_w // 128
    mesh = plsc.VectorSubcoreMesh(core_axis_name="c", subcore_axis_name="s")

    @functools.partial(
        pl.kernel,
        mesh=mesh,
        compiler_params=pltpu.CompilerParams(use_tc_tiling_on_sc=False),
        out_type=jax.ShapeDtypeStruct((_N, _D), jnp.float32),
        scratch_types=[
            pltpu.VMEM((chunks, 128), jnp.int32),
            pltpu.VMEM((chunks, 128, _D), jnp.float32),
            pltpu.SemaphoreType.DMA,
        ],
    )
    def gather_kernel(idx_hbm, table_hbm, out_hbm, idx_v, rows_v, sem):
        wid = lax.axis_index("s") * nc + lax.axis_index("c")
        base = wid * b_per_w
        for k in range(chunks):
            off = base + k * 128
            pltpu.sync_copy(idx_hbm.at[pl.ds(off, 128)], idx_v.at[k])
            pltpu.async_copy(table_hbm.at[idx_v.at[k]], rows_v.at[k], sem).wait()
            pltpu.sync_copy(rows_v.at[k], out_hbm.at[pl.ds(off, 128)])

    return gather_kernel(idx, table)


def kernel(x, embed):
    x = x.astype(jnp.float32)
    h = x.shape[0]
    x2d = x.reshape(_N, _D)
    e2d = embed.reshape(_C, _D)
    dist2d, ind2d = _dist_argmin(x2d, e2d.T)
    ind_flat = ind2d.reshape(_N)
    quant2d = _sc_gather(e2d, ind_flat)
    quantize = quant2d.reshape(h, _N, _D)
    embed_ind = ind_flat.reshape(h, _N)
    dist_u = dist2d.reshape(h, _N, _C)
    return (quantize, embed_ind, dist_u)


# speedup vs baseline: 2.4544x; 1.0035x over previous
import functools

import jax
import jax.numpy as jnp
from jax import lax
from jax.experimental import pallas as pl
from jax.experimental.pallas import tpu as pltpu
from jax.experimental.pallas import tpu_sc as plsc

_N = 8192
_C = 8192
_D = 32
_TN = 512
_TC = 8192
_NI = _N // _TN
_NJ = _C // _TC
_INT_MAX = jnp.iinfo(jnp.int32).max
_NJ_C = _TC // 128


_TG = 256


def _dist_body(x_ref, et_ref, dist_ref, ind_ref, m_sc, a_sc):
    j = pl.program_id(1)
    xs = x_ref[...]
    es = et_ref[...]
    x2 = jnp.sum(xs * xs, axis=1, keepdims=True)
    y2 = jnp.sum(es * es, axis=0, keepdims=True)

    @pl.when(j == 0)
    def _():
        m_sc[...] = jnp.full((_TN, 128), jnp.inf, jnp.float32)
        a_sc[...] = jnp.zeros((_TN, 128), jnp.int32)

    lane = lax.broadcasted_iota(jnp.int32, (_TN, 128), 1)
    m_run = m_sc[...]
    a_run = a_sc[...]

    for g in range(_TC // _TG):
        es_g = es[:, g * _TG:(g + 1) * _TG]
        xy = lax.dot_general(
            xs, es_g, (((1,), (0,)), ((), ())),
            preferred_element_type=jnp.float32,
        )
        y2_g = y2[:, g * _TG:(g + 1) * _TG]
        sq = (x2 + y2_g) + xy * -2.0
        s = jnp.sqrt(jnp.maximum(sq, 0.0))
        dist_ref[:, g * _TG:(g + 1) * _TG] = -s
        for h in range(_TG // 128):
            s_h = s[:, h * 128:(h + 1) * 128]
            upd = s_h < m_run
            a_run = jnp.where(upd, jnp.int32(j * _NJ_C + g * (_TG // 128) + h), a_run)
            m_run = jnp.minimum(m_run, s_h)

    m_sc[...] = m_run
    a_sc[...] = a_run

    @pl.when(j == _NJ - 1)
    def _():
        col = a_run * 128 + lane
        rmin = jnp.min(m_run, axis=1, keepdims=True)
        cand = jnp.where(m_run == rmin, col, _INT_MAX)
        ind_ref[...] = jnp.min(cand, axis=1, keepdims=True)


def _dist_argmin(x2d, embed_t):
    return pl.pallas_call(
        _dist_body,
        grid=(_NI, _NJ),
        in_specs=[
            pl.BlockSpec((_TN, _D), lambda i, j: (i, 0)),
            pl.BlockSpec((_D, _TC), lambda i, j: (0, j)),
        ],
        out_specs=[
            pl.BlockSpec((_TN, _TC), lambda i, j: (i, j)),
            pl.BlockSpec((_TN, 1), lambda i, j: (i, 0)),
        ],
        out_shape=[
            jax.ShapeDtypeStruct((_N, _C), jnp.float32),
            jax.ShapeDtypeStruct((_N, 1), jnp.int32),
        ],
        scratch_shapes=[
            pltpu.VMEM((_TN, 128), jnp.float32),
            pltpu.VMEM((_TN, 128), jnp.int32),
        ],
    )(x2d, embed_t)


def _sc_gather(table, idx):
    info = plsc.get_sparse_core_info()
    nc, ns = info.num_cores, info.num_subcores
    nw = nc * ns
    b_per_w = _N // nw
    chunks = b_per_w // 128
    mesh = plsc.VectorSubcoreMesh(core_axis_name="c", subcore_axis_name="s")

    @functools.partial(
        pl.kernel,
        mesh=mesh,
        compiler_params=pltpu.CompilerParams(use_tc_tiling_on_sc=False),
        out_type=jax.ShapeDtypeStruct((_N, _D), jnp.float32),
        scratch_types=[
            pltpu.VMEM((chunks, 128), jnp.int32),
            pltpu.VMEM((chunks, 128, _D), jnp.float32),
            pltpu.SemaphoreType.DMA,
        ],
    )
    def gather_kernel(idx_hbm, table_hbm, out_hbm, idx_v, rows_v, sem):
        wid = lax.axis_index("s") * nc + lax.axis_index("c")
        base = wid * b_per_w
        for k in range(chunks):
            pltpu.sync_copy(idx_hbm.at[pl.ds(base + k * 128, 128)], idx_v.at[k])
        copies = [
            pltpu.async_copy(table_hbm.at[idx_v.at[k]], rows_v.at[k], sem)
            for k in range(chunks)
        ]
        for c in copies:
            c.wait()
        for k in range(chunks):
            pltpu.sync_copy(rows_v.at[k], out_hbm.at[pl.ds(base + k * 128, 128)])

    return gather_kernel(idx, table)


def kernel(x, embed):
    x = x.astype(jnp.float32)
    h = x.shape[0]
    x2d = x.reshape(_N, _D)
    e2d = embed.reshape(_C, _D)
    dist2d, ind2d = _dist_argmin(x2d, e2d.T)
    ind_flat = ind2d.reshape(_N)
    quant2d = _sc_gather(e2d, ind_flat)
    quantize = quant2d.reshape(h, _N, _D)
    embed_ind = ind_flat.reshape(h, _N)
    dist_u = dist2d.reshape(h, _N, _C)
    return (quantize, embed_ind, dist_u)
